# baseline (speedup 1.0000x reference)
import functools

import jax
import jax.numpy as jnp
from jax import lax
from jax.experimental import pallas as pl
from jax.experimental.pallas import tpu as pltpu
from jax.experimental.pallas import tpu_sc as plsc

NUM_EMBEDDINGS = 100000
EMBEDDING_DIM = 128

_INFO = plsc.get_sparse_core_info()
_NW = _INFO.num_cores * _INFO.num_subcores

_IDX = 128
_SUPER = 256
_NBUF = 3
_LOOK = 2


def _embed_kernel(b_per_w, n_sup, table_hbm, ids_hbm, out_hbm,
                  idx_v, rows_v, gsem, wsem):
    wid = lax.axis_index("s") * _INFO.num_cores + lax.axis_index("c")
    base = wid * b_per_w
    pltpu.sync_copy(ids_hbm.at[pl.ds(base, b_per_w)], idx_v)

    def g_start(i, b):
        for h in range(_SUPER // _IDX):
            pltpu.make_async_copy(
                table_hbm.at[idx_v.at[pl.ds(i * _SUPER + h * _IDX, _IDX)]],
                rows_v.at[b, pl.ds(h * _IDX, _IDX)], gsem.at[b]).start()

    def g_wait(i, b):
        pltpu.make_async_copy(
            table_hbm.at[idx_v.at[pl.ds(i * _SUPER, _SUPER)]],
            rows_v.at[b], gsem.at[b]).wait()

    def w_copy(i, b):
        return pltpu.make_async_copy(
            rows_v.at[b], out_hbm.at[pl.ds(base + i * _SUPER, _SUPER)],
            wsem.at[b])

    for b in range(_LOOK):
        g_start(b, b)

    n_groups = (n_sup - 1) // _NBUF

    def group(g, _):
        i0 = g * _NBUF
        for b in range(_NBUF):
            i = i0 + b
            g_wait(i, b)
            w_copy(i, b).start()
            k = i + _LOOK
            kb = (b + _LOOK) % _NBUF

            def issue_next(kk):
                w_copy(kk - _NBUF, kb).wait()
                g_start(kk, kb)

            if b + _LOOK < _NBUF:
                pl.when(g > 0)(lambda: issue_next(k))
                pl.when(g == 0)(lambda: g_start(k, kb))
            else:
                pl.when(k < n_sup)(lambda: issue_next(k))
        return 0

    lax.fori_loop(0, n_groups, group, 0)

    for r in range(n_groups * _NBUF, n_sup):
        b = r % _NBUF
        g_wait(r, b)
        w_copy(r, b).start()

    for r in range(n_sup - _NBUF, n_sup):
        w_copy(r, r % _NBUF).wait()


def kernel(input_ids, table):
    B = input_ids.shape[0] * input_ids.shape[1]
    assert B % (_NW * _SUPER) == 0
    b_per_w = B // _NW
    n_sup = b_per_w // _SUPER
    ids_flat = input_ids.reshape(B).astype(jnp.int32)

    mesh = plsc.VectorSubcoreMesh(core_axis_name="c", subcore_axis_name="s")
    run = pl.kernel(
        functools.partial(_embed_kernel, b_per_w, n_sup),
        mesh=mesh,
        out_type=jax.ShapeDtypeStruct((B, EMBEDDING_DIM), jnp.float32),
        scratch_types=[
            pltpu.VMEM((b_per_w,), jnp.int32),
            pltpu.VMEM((_NBUF, _SUPER, EMBEDDING_DIM), jnp.float32),
            pltpu.SemaphoreType.DMA((_NBUF,)),
            pltpu.SemaphoreType.DMA((_NBUF,)),
        ],
    )
    out = run(table, ids_flat)
    return out.reshape(input_ids.shape[0], input_ids.shape[1], EMBEDDING_DIM)
